# baseline (speedup 1.0000x reference)
import functools

import jax
import jax.numpy as jnp
from jax.experimental import pallas as pl
from jax.experimental.pallas import tpu as pltpu


def _proj_kernel(x_ref, w_ref, b_ref, xl_ref, xr_ref, *, f):
    y = jnp.dot(x_ref[...], w_ref[...], preferred_element_type=jnp.float32)
    xl_ref[...] = y[:, :f]
    xr_ref[...] = y[:, f:] + b_ref[...]


def _agg_part_kernel(ctile_ref, cfirst_ref, clast_ref, srcp_ref,
                     ldst_ref, xl_ref, part_ref, pcnt_ref,
                     msgs_ref, acc_ref, dcnt_ref, *, ec, ti, nch, dw):
    k = pl.program_id(0)
    j = pl.program_id(1)
    c = k * nch + j

    @pl.when(cfirst_ref[c] == 1)
    def _():
        acc_ref[...] = jnp.zeros_like(acc_ref)
        dcnt_ref[...] = jnp.zeros_like(dcnt_ref)

    base = c * ec
    for mi in range(ec):
        idx = srcp_ref[base + mi]
        msgs_ref[pl.ds(mi, 1), :] = xl_ref[pl.ds(idx, 1), :]

    rows = jax.lax.broadcasted_iota(jnp.int32, (ti, ec), 0)
    oh = (rows == ldst_ref[...].reshape(1, ec)).astype(jnp.bfloat16)
    acc_ref[...] += jnp.dot(oh, msgs_ref[...].astype(jnp.bfloat16),
                            preferred_element_type=jnp.float32)
    dcnt_ref[...] += jnp.sum(oh, axis=1, keepdims=True).astype(jnp.float32)

    @pl.when(clast_ref[c] == 1)
    def _():
        part_ref[0, :, :] = acc_ref[:, :dw]
        pcnt_ref[0, :, :] = dcnt_ref[...]


def _comb_mid_kernel(p_ref, c_ref, xr_ref, w2_ref, b2_ref,
                     xl2_ref, xr2_ref, *, f):
    dinv = 1.0 / jnp.maximum(c_ref[0] + c_ref[1], 1.0)
    h = jnp.maximum((p_ref[0] + p_ref[1]) * dinv + xr_ref[...], 0.0)
    y2 = jnp.dot(h.astype(jnp.bfloat16), w2_ref[...],
                 preferred_element_type=jnp.float32)
    xl2_ref[...] = y2[:, :f]
    xr2_ref[...] = y2[:, f:] + b2_ref[...]


def _comb_out_kernel(p_ref, c_ref, xr_ref, out_ref):
    dinv = 1.0 / jnp.maximum(c_ref[0] + c_ref[1], 1.0)
    z = (p_ref[0] + p_ref[1]) * dinv + xr_ref[...]
    m = jnp.max(z, axis=-1, keepdims=True)
    lse = jnp.log(jnp.sum(jnp.exp(z - m), axis=-1, keepdims=True)) + m
    out_ref[...] = z - lse


def _sparse_agg(xl, ctile, cfirst, clast, srcp, ldst3, *, n, ec, ti, nch, dw,
                vlim):
    f = xl.shape[1]
    return pl.pallas_call(
        functools.partial(_agg_part_kernel, ec=ec, ti=ti, nch=nch, dw=dw),
        out_shape=(jax.ShapeDtypeStruct((2, n, dw), jnp.float32),
                   jax.ShapeDtypeStruct((2, n, 1), jnp.float32)),
        grid_spec=pltpu.PrefetchScalarGridSpec(
            num_scalar_prefetch=4,
            grid=(2, nch),
            in_specs=[
                pl.BlockSpec((1, 1, ec),
                             lambda k, j, ct, cf, cl, sp: (k * nch + j, 0, 0)),
                pl.BlockSpec((n, f), lambda k, j, ct, cf, cl, sp: (0, 0)),
            ],
            out_specs=(
                pl.BlockSpec((1, ti, dw),
                             lambda k, j, ct, cf, cl, sp: (k, ct[k * nch + j], 0)),
                pl.BlockSpec((1, ti, 1),
                             lambda k, j, ct, cf, cl, sp: (k, ct[k * nch + j], 0)),
            ),
            scratch_shapes=[pltpu.VMEM((ec, f), jnp.float32),
                            pltpu.VMEM((ti, f), jnp.float32),
                            pltpu.VMEM((ti, 1), jnp.float32)],
        ),
        compiler_params=pltpu.CompilerParams(
            dimension_semantics=("parallel", "arbitrary"),
            vmem_limit_bytes=vlim),
    )(ctile, cfirst, clast, srcp, ldst3, xl)


def kernel(x, edge_index, w1_l, w1_r, b1, w2_l, w2_r, b2):
    n, din = x.shape
    dh = w1_l.shape[0]
    dout = w2_l.shape[0]
    e = edge_index.shape[1]

    ti = 256
    ec = 256
    nt = n // ti
    esz0 = e // 2
    esz1 = e - esz0
    nch = (esz1 + ec - 1) // ec + nt
    spc = nch * ec

    src, dst = edge_index[0], edge_index[1]
    src = src.astype(jnp.int32)
    dst = dst.astype(jnp.int32)

    tile = dst // ti
    ldst_e = dst - tile * ti
    core = (jnp.arange(e, dtype=jnp.int32) >= esz0).astype(jnp.int32)

    t_iota = jnp.arange(nt, dtype=jnp.int32)
    cnt0 = jnp.sum(tile[:esz0, None] == t_iota[None, :], axis=0,
                   dtype=jnp.int32)
    cnt1 = jnp.sum(tile[esz0:, None] == t_iota[None, :], axis=0,
                   dtype=jnp.int32)
    cnt2 = jnp.stack([cnt0, cnt1])
    pcnt2 = jnp.maximum((cnt2 + ec - 1) // ec, 1) * ec
    pstart2 = jnp.concatenate(
        [jnp.zeros((2, 1), jnp.int32),
         jnp.cumsum(pcnt2, axis=1).astype(jnp.int32)], axis=1)

    b1k = ti + ec + 1
    b2k = (nt + 2) * b1k

    key_real = core * b2k + tile * b1k + ldst_e
    val_real = ((ldst_e + 1) << 14) | src

    pads = []
    for k in range(2):
        ncand = spc - (esz0 if k == 0 else esz1)
        ci = jnp.arange(ncand, dtype=jnp.int32)
        t = ci // ec
        i = ci - t * ec
        need = jnp.repeat(pcnt2[k] - cnt2[k], ec)
        need = jnp.concatenate(
            [need, jnp.zeros((max(ncand - nt * ec, 0),), jnp.int32)])[:ncand]
        valid = (t < nt) & (i < need)
        key = jnp.where(valid, k * b2k + t * b1k + ti + i,
                        k * b2k + (nt + 1) * b1k + i % b1k)
        pads.append(key)
    keys = jnp.concatenate([key_real, pads[0], pads[1]])
    vals = jnp.concatenate([val_real, jnp.zeros((2 * spc - e,), jnp.int32)])
    _, vals_s = jax.lax.sort_key_val(keys, vals)

    srcp = vals_s & ((1 << 14) - 1)
    ldst3 = ((vals_s >> 14) - 1).reshape(2 * nch, 1, ec)

    jslot = (jnp.arange(nch, dtype=jnp.int32) * ec)[None, :, None]
    ctile = jnp.sum(pstart2[:, None, :] <= jslot, axis=2,
                    dtype=jnp.int32) - 1
    ctile = jnp.clip(ctile, 0, nt - 1)
    chg = (ctile[:, 1:] != ctile[:, :-1]).astype(jnp.int32)
    one = jnp.ones((2, 1), jnp.int32)
    cfirst = jnp.concatenate([one, chg], axis=1).reshape(-1)
    clast = jnp.concatenate([chg, one], axis=1).reshape(-1)
    ctile = ctile.reshape(-1)

    cd = jnp.bfloat16
    w1 = jnp.concatenate([w1_l.T, w1_r.T], axis=1).astype(cd)
    b1r = b1.reshape(1, dh).astype(jnp.float32)
    f2 = dh
    w2 = jnp.concatenate([jnp.pad(w2_l.T, ((0, 0), (0, f2 - dout))),
                          w2_r.T], axis=1).astype(cd)
    b2r = b2.reshape(1, dout).astype(jnp.float32)

    tp = 512
    xl1, xr1 = pl.pallas_call(
        functools.partial(_proj_kernel, f=dh),
        out_shape=(jax.ShapeDtypeStruct((n, dh), jnp.float32),
                   jax.ShapeDtypeStruct((n, dh), jnp.float32)),
        grid=(n // tp,),
        in_specs=[pl.BlockSpec((tp, din), lambda i: (i, 0)),
                  pl.BlockSpec((din, 2 * dh), lambda i: (0, 0)),
                  pl.BlockSpec((1, dh), lambda i: (0, 0))],
        out_specs=(pl.BlockSpec((tp, dh), lambda i: (i, 0)),
                   pl.BlockSpec((tp, dh), lambda i: (i, 0))),
        compiler_params=pltpu.CompilerParams(
            dimension_semantics=("parallel",)),
    )(x.astype(cd), w1, b1r)

    vlim = 48 * 1024 * 1024
    part1, pcn1 = _sparse_agg(xl1, ctile, cfirst, clast, srcp, ldst3,
                              n=n, ec=ec, ti=ti, nch=nch, dw=dh, vlim=vlim)
    tc = 512
    xl2, xr2 = pl.pallas_call(
        functools.partial(_comb_mid_kernel, f=f2),
        out_shape=(jax.ShapeDtypeStruct((n, f2), jnp.float32),
                   jax.ShapeDtypeStruct((n, dout), jnp.float32)),
        grid=(n // tc,),
        in_specs=[pl.BlockSpec((2, tc, dh), lambda i: (0, i, 0)),
                  pl.BlockSpec((2, tc, 1), lambda i: (0, i, 0)),
                  pl.BlockSpec((tc, dh), lambda i: (i, 0)),
                  pl.BlockSpec((dh, f2 + dout), lambda i: (0, 0)),
                  pl.BlockSpec((1, dout), lambda i: (0, 0))],
        out_specs=(pl.BlockSpec((tc, f2), lambda i: (i, 0)),
                   pl.BlockSpec((tc, dout), lambda i: (i, 0))),
        compiler_params=pltpu.CompilerParams(
            dimension_semantics=("parallel",)),
    )(part1, pcn1, xr1, w2, b2r)

    part2, pcn2 = _sparse_agg(xl2, ctile, cfirst, clast, srcp, ldst3,
                              n=n, ec=ec, ti=ti, nch=nch, dw=dout, vlim=vlim)
    out = pl.pallas_call(
        _comb_out_kernel,
        out_shape=jax.ShapeDtypeStruct((n, dout), jnp.float32),
        grid=(n // tc,),
        in_specs=[pl.BlockSpec((2, tc, dout), lambda i: (0, i, 0)),
                  pl.BlockSpec((2, tc, 1), lambda i: (0, i, 0)),
                  pl.BlockSpec((tc, dout), lambda i: (i, 0))],
        out_specs=pl.BlockSpec((tc, dout), lambda i: (i, 0)),
        compiler_params=pltpu.CompilerParams(
            dimension_semantics=("parallel",)),
    )(part2, pcn2, xr2)

    return out


# speedup vs baseline: 1.0969x; 1.0969x over previous
import functools

import jax
import jax.numpy as jnp
from jax.experimental import pallas as pl
from jax.experimental.pallas import tpu as pltpu


def _proj_kernel(x_ref, w_ref, b_ref, xl_ref, xr_ref, *, f):
    y = jnp.dot(x_ref[...], w_ref[...], preferred_element_type=jnp.float32)
    xl_ref[...] = y[:, :f]
    xr_ref[...] = y[:, f:] + b_ref[...]


def _agg_part_kernel(ctile_ref, cfirst_ref, clast_ref, cactive_ref, srcp_ref,
                     ldst_ref, xl_ref, part_ref, pcnt_ref,
                     msgs_ref, acc_ref, dcnt_ref, *, ec, ti, nch, dw):
    k = pl.program_id(0)
    j = pl.program_id(1)
    c = k * nch + j

    @pl.when(cfirst_ref[c] == 1)
    def _():
        acc_ref[...] = jnp.zeros_like(acc_ref)
        dcnt_ref[...] = jnp.zeros_like(dcnt_ref)

    @pl.when(cactive_ref[c] == 1)
    def _():
        base = c * ec
        for mi in range(ec):
            idx = srcp_ref[base + mi]
            msgs_ref[pl.ds(mi, 1), :] = xl_ref[pl.ds(idx, 1), :]

        rows = jax.lax.broadcasted_iota(jnp.int32, (ti, ec), 0)
        oh = (rows == ldst_ref[...].reshape(1, ec)).astype(jnp.bfloat16)
        acc_ref[...] += jnp.dot(oh, msgs_ref[...].astype(jnp.bfloat16),
                                preferred_element_type=jnp.float32)
        dcnt_ref[...] += jnp.sum(oh, axis=1,
                                 keepdims=True).astype(jnp.float32)

    @pl.when(clast_ref[c] == 1)
    def _():
        part_ref[0, :, :] = acc_ref[:, :dw]
        pcnt_ref[0, :, :] = dcnt_ref[...]


def _comb_mid_kernel(p_ref, c_ref, xr_ref, w2_ref, b2_ref,
                     xl2_ref, xr2_ref, *, f):
    dinv = 1.0 / jnp.maximum(c_ref[0] + c_ref[1], 1.0)
    h = jnp.maximum((p_ref[0] + p_ref[1]) * dinv + xr_ref[...], 0.0)
    y2 = jnp.dot(h.astype(jnp.bfloat16), w2_ref[...],
                 preferred_element_type=jnp.float32)
    xl2_ref[...] = y2[:, :f]
    xr2_ref[...] = y2[:, f:] + b2_ref[...]


def _comb_out_kernel(p_ref, c_ref, xr_ref, out_ref):
    dinv = 1.0 / jnp.maximum(c_ref[0] + c_ref[1], 1.0)
    z = (p_ref[0] + p_ref[1]) * dinv + xr_ref[...]
    m = jnp.max(z, axis=-1, keepdims=True)
    lse = jnp.log(jnp.sum(jnp.exp(z - m), axis=-1, keepdims=True)) + m
    out_ref[...] = z - lse


def _sparse_agg(xl, ctile, cfirst, clast, cactive, srcp, ldst3, *, n, ec, ti, nch, dw,
                vlim):
    f = xl.shape[1]
    return pl.pallas_call(
        functools.partial(_agg_part_kernel, ec=ec, ti=ti, nch=nch, dw=dw),
        out_shape=(jax.ShapeDtypeStruct((2, n, dw), jnp.float32),
                   jax.ShapeDtypeStruct((2, n, 1), jnp.float32)),
        grid_spec=pltpu.PrefetchScalarGridSpec(
            num_scalar_prefetch=5,
            grid=(2, nch),
            in_specs=[
                pl.BlockSpec((1, 1, ec),
                             lambda k, j, ct, cf, cl, ca, sp: (k * nch + j, 0, 0)),
                pl.BlockSpec((n, f), lambda k, j, ct, cf, cl, ca, sp: (0, 0)),
            ],
            out_specs=(
                pl.BlockSpec((1, ti, dw),
                             lambda k, j, ct, cf, cl, ca, sp: (k, ct[k * nch + j], 0)),
                pl.BlockSpec((1, ti, 1),
                             lambda k, j, ct, cf, cl, ca, sp: (k, ct[k * nch + j], 0)),
            ),
            scratch_shapes=[pltpu.VMEM((ec, f), jnp.float32),
                            pltpu.VMEM((ti, f), jnp.float32),
                            pltpu.VMEM((ti, 1), jnp.float32)],
        ),
        compiler_params=pltpu.CompilerParams(
            dimension_semantics=("parallel", "arbitrary"),
            vmem_limit_bytes=vlim),
    )(ctile, cfirst, clast, cactive, srcp, ldst3, xl)


def kernel(x, edge_index, w1_l, w1_r, b1, w2_l, w2_r, b2):
    n, din = x.shape
    dh = w1_l.shape[0]
    dout = w2_l.shape[0]
    e = edge_index.shape[1]

    ti = 256
    ec = 256
    nt = n // ti
    esz0 = e // 2
    esz1 = e - esz0
    nch = (esz1 + ec - 1) // ec + nt
    spc = nch * ec

    src, dst = edge_index[0], edge_index[1]
    src = src.astype(jnp.int32)
    dst = dst.astype(jnp.int32)

    tile = dst // ti
    ldst_e = dst - tile * ti
    core = (jnp.arange(e, dtype=jnp.int32) >= esz0).astype(jnp.int32)

    t_iota = jnp.arange(nt, dtype=jnp.int32)
    cnt0 = jnp.sum(tile[:esz0, None] == t_iota[None, :], axis=0,
                   dtype=jnp.int32)
    cnt1 = jnp.sum(tile[esz0:, None] == t_iota[None, :], axis=0,
                   dtype=jnp.int32)
    cnt2 = jnp.stack([cnt0, cnt1])
    pcnt2 = jnp.maximum((cnt2 + ec - 1) // ec, 1) * ec
    pstart2 = jnp.concatenate(
        [jnp.zeros((2, 1), jnp.int32),
         jnp.cumsum(pcnt2, axis=1).astype(jnp.int32)], axis=1)

    lb = 14
    key_real = (core << 30) | (tile << 23) | (ldst_e << lb) | src

    pads = []
    for k in range(2):
        ncand = spc - (esz0 if k == 0 else esz1)
        ci = jnp.arange(ncand, dtype=jnp.int32)
        t = ci // ec
        i = ci - t * ec
        need = jnp.repeat(pcnt2[k] - cnt2[k], ec)
        need = jnp.concatenate(
            [need, jnp.zeros((max(ncand - nt * ec, 0),), jnp.int32)])[:ncand]
        valid = (t < nt) & (i < need)
        key = jnp.where(valid,
                        (k << 30) | (t << 23) | (ti << lb) | i,
                        (k << 30) | (nt << 23) | (ti << lb)
                        | (ci & ((1 << lb) - 1)))
        pads.append(key)
    keys_s = jnp.sort(jnp.concatenate([key_real, pads[0], pads[1]]))

    srcp = keys_s & ((1 << lb) - 1)
    ldst3 = ((keys_s >> lb) & 511).reshape(2 * nch, 1, ec)

    jslot = (jnp.arange(nch, dtype=jnp.int32) * ec)[None, :, None]
    ctile2 = jnp.sum(pstart2[:, None, :] <= jslot, axis=2,
                     dtype=jnp.int32) - 1
    ctile2 = jnp.clip(ctile2, 0, nt - 1)
    chg = (ctile2[:, 1:] != ctile2[:, :-1]).astype(jnp.int32)
    one = jnp.ones((2, 1), jnp.int32)
    cfirst = jnp.concatenate([one, chg], axis=1).reshape(-1)
    clast = jnp.concatenate([chg, one], axis=1).reshape(-1)
    ctile = ctile2.reshape(-1)
    limit = jnp.take_along_axis(pstart2[:, :nt] + cnt2, jnp.clip(ctile2, 0, nt - 1),
                                axis=1)
    cactive = ((jnp.arange(nch, dtype=jnp.int32) * ec)[None, :] < limit
               ).astype(jnp.int32).reshape(-1)

    cd = jnp.bfloat16
    w1 = jnp.concatenate([w1_l.T, w1_r.T], axis=1).astype(cd)
    b1r = b1.reshape(1, dh).astype(jnp.float32)
    f2 = dh
    w2 = jnp.concatenate([jnp.pad(w2_l.T, ((0, 0), (0, f2 - dout))),
                          w2_r.T], axis=1).astype(cd)
    b2r = b2.reshape(1, dout).astype(jnp.float32)

    tp = 512
    xl1, xr1 = pl.pallas_call(
        functools.partial(_proj_kernel, f=dh),
        out_shape=(jax.ShapeDtypeStruct((n, dh), jnp.float32),
                   jax.ShapeDtypeStruct((n, dh), jnp.float32)),
        grid=(n // tp,),
        in_specs=[pl.BlockSpec((tp, din), lambda i: (i, 0)),
                  pl.BlockSpec((din, 2 * dh), lambda i: (0, 0)),
                  pl.BlockSpec((1, dh), lambda i: (0, 0))],
        out_specs=(pl.BlockSpec((tp, dh), lambda i: (i, 0)),
                   pl.BlockSpec((tp, dh), lambda i: (i, 0))),
        compiler_params=pltpu.CompilerParams(
            dimension_semantics=("parallel",)),
    )(x.astype(cd), w1, b1r)

    vlim = 48 * 1024 * 1024
    part1, pcn1 = _sparse_agg(xl1, ctile, cfirst, clast, cactive, srcp, ldst3,
                              n=n, ec=ec, ti=ti, nch=nch, dw=dh, vlim=vlim)
    tc = 512
    xl2, xr2 = pl.pallas_call(
        functools.partial(_comb_mid_kernel, f=f2),
        out_shape=(jax.ShapeDtypeStruct((n, f2), jnp.float32),
                   jax.ShapeDtypeStruct((n, dout), jnp.float32)),
        grid=(n // tc,),
        in_specs=[pl.BlockSpec((2, tc, dh), lambda i: (0, i, 0)),
                  pl.BlockSpec((2, tc, 1), lambda i: (0, i, 0)),
                  pl.BlockSpec((tc, dh), lambda i: (i, 0)),
                  pl.BlockSpec((dh, f2 + dout), lambda i: (0, 0)),
                  pl.BlockSpec((1, dout), lambda i: (0, 0))],
        out_specs=(pl.BlockSpec((tc, f2), lambda i: (i, 0)),
                   pl.BlockSpec((tc, dout), lambda i: (i, 0))),
        compiler_params=pltpu.CompilerParams(
            dimension_semantics=("parallel",)),
    )(part1, pcn1, xr1, w2, b2r)

    part2, pcn2 = _sparse_agg(xl2, ctile, cfirst, clast, cactive, srcp, ldst3,
                              n=n, ec=ec, ti=ti, nch=nch, dw=dout, vlim=vlim)
    out = pl.pallas_call(
        _comb_out_kernel,
        out_shape=jax.ShapeDtypeStruct((n, dout), jnp.float32),
        grid=(n // tc,),
        in_specs=[pl.BlockSpec((2, tc, dout), lambda i: (0, i, 0)),
                  pl.BlockSpec((2, tc, 1), lambda i: (0, i, 0)),
                  pl.BlockSpec((tc, dout), lambda i: (i, 0))],
        out_specs=pl.BlockSpec((tc, dout), lambda i: (i, 0)),
        compiler_params=pltpu.CompilerParams(
            dimension_semantics=("parallel",)),
    )(part2, pcn2, xr2)

    return out


# speedup vs baseline: 1.1225x; 1.0233x over previous
import functools

import jax
import jax.numpy as jnp
from jax.experimental import pallas as pl
from jax.experimental.pallas import tpu as pltpu


def _proj_kernel(x_ref, w_ref, b_ref, xl_ref, xr_ref, *, f):
    y = jnp.dot(x_ref[...], w_ref[...], preferred_element_type=jnp.float32)
    xl_ref[...] = y[:, :f]
    xr_ref[...] = y[:, f:] + b_ref[...]


def _agg_part_kernel(ctile_ref, cfirst_ref, clast_ref, cactive_ref, srcp_ref,
                     ldst_ref, xl_ref, *refs, ec, ti, nch, dw, with_cnt):
    if with_cnt:
        part_ref, pcnt_ref, msgs_ref, acc_ref, dcnt_ref = refs
    else:
        part_ref, msgs_ref, acc_ref = refs
        pcnt_ref = dcnt_ref = None
    k = pl.program_id(0)
    j = pl.program_id(1)
    c = k * nch + j

    @pl.when(cfirst_ref[c] == 1)
    def _():
        acc_ref[...] = jnp.zeros_like(acc_ref)
        if with_cnt:
            dcnt_ref[...] = jnp.zeros_like(dcnt_ref)

    @pl.when(cactive_ref[c] == 1)
    def _():
        base = c * ec
        for mi in range(ec):
            idx = srcp_ref[base + mi]
            msgs_ref[pl.ds(mi, 1), :] = xl_ref[pl.ds(idx, 1), :]

        rows = jax.lax.broadcasted_iota(jnp.int32, (ti, ec), 0)
        oh = (rows == ldst_ref[...].reshape(1, ec)).astype(jnp.bfloat16)
        acc_ref[...] += jnp.dot(oh, msgs_ref[...].astype(jnp.bfloat16),
                                preferred_element_type=jnp.float32)
        if with_cnt:
            dcnt_ref[...] += jnp.sum(oh, axis=1,
                                     keepdims=True).astype(jnp.float32)

    @pl.when(clast_ref[c] == 1)
    def _():
        part_ref[0, :, :] = acc_ref[:, :dw].astype(part_ref.dtype)
        if with_cnt:
            pcnt_ref[0, :, :] = dcnt_ref[...]


def _comb_mid_kernel(p_ref, c_ref, xr_ref, w2_ref, b2_ref,
                     xl2_ref, xr2_ref, *, f):
    dinv = 1.0 / jnp.maximum(c_ref[0] + c_ref[1], 1.0)
    psum = p_ref[0].astype(jnp.float32) + p_ref[1].astype(jnp.float32)
    h = jnp.maximum(psum * dinv + xr_ref[...], 0.0)
    y2 = jnp.dot(h.astype(jnp.bfloat16), w2_ref[...],
                 preferred_element_type=jnp.float32)
    xl2_ref[...] = y2[:, :f]
    xr2_ref[...] = y2[:, f:] + b2_ref[...]


def _comb_out_kernel(p_ref, c_ref, xr_ref, out_ref):
    dinv = 1.0 / jnp.maximum(c_ref[0] + c_ref[1], 1.0)
    z = (p_ref[0].astype(jnp.float32)
         + p_ref[1].astype(jnp.float32)) * dinv + xr_ref[...]
    m = jnp.max(z, axis=-1, keepdims=True)
    lse = jnp.log(jnp.sum(jnp.exp(z - m), axis=-1, keepdims=True)) + m
    out_ref[...] = z - lse


def _sparse_agg(xl, ctile, cfirst, clast, cactive, srcp, ldst3, *, n, ec, ti, nch, dw,
                vlim, with_cnt):
    f = xl.shape[1]
    out_shape = [jax.ShapeDtypeStruct((2, n, dw), jnp.bfloat16)]
    out_specs = [pl.BlockSpec((1, ti, dw),
                              lambda k, j, ct, cf, cl, ca, sp: (k, ct[k * nch + j], 0))]
    scratch = [pltpu.VMEM((ec, f), jnp.float32),
               pltpu.VMEM((ti, f), jnp.float32)]
    if with_cnt:
        out_shape.append(jax.ShapeDtypeStruct((2, n, 1), jnp.float32))
        out_specs.append(pl.BlockSpec(
            (1, ti, 1), lambda k, j, ct, cf, cl, ca, sp: (k, ct[k * nch + j], 0)))
        scratch.append(pltpu.VMEM((ti, 1), jnp.float32))
    return pl.pallas_call(
        functools.partial(_agg_part_kernel, ec=ec, ti=ti, nch=nch, dw=dw,
                          with_cnt=with_cnt),
        out_shape=tuple(out_shape),
        grid_spec=pltpu.PrefetchScalarGridSpec(
            num_scalar_prefetch=5,
            grid=(2, nch),
            in_specs=[
                pl.BlockSpec((1, 1, ec),
                             lambda k, j, ct, cf, cl, ca, sp: (k * nch + j, 0, 0)),
                pl.BlockSpec((n, f), lambda k, j, ct, cf, cl, ca, sp: (0, 0)),
            ],
            out_specs=tuple(out_specs),
            scratch_shapes=scratch,
        ),
        compiler_params=pltpu.CompilerParams(
            dimension_semantics=("parallel", "arbitrary"),
            vmem_limit_bytes=vlim),
    )(ctile, cfirst, clast, cactive, srcp, ldst3, xl)


def kernel(x, edge_index, w1_l, w1_r, b1, w2_l, w2_r, b2):
    n, din = x.shape
    dh = w1_l.shape[0]
    dout = w2_l.shape[0]
    e = edge_index.shape[1]

    ti = 256
    ec = 256
    nt = n // ti
    esz0 = e // 2
    esz1 = e - esz0
    nch = (esz1 + ec - 1) // ec + nt
    spc = nch * ec

    src, dst = edge_index[0], edge_index[1]
    src = src.astype(jnp.int32)
    dst = dst.astype(jnp.int32)

    tile = dst // ti
    ldst_e = dst - tile * ti
    core = (jnp.arange(e, dtype=jnp.int32) >= esz0).astype(jnp.int32)

    t_iota = jnp.arange(nt, dtype=jnp.int32)
    cnt0 = jnp.sum(tile[:esz0, None] == t_iota[None, :], axis=0,
                   dtype=jnp.int32)
    cnt1 = jnp.sum(tile[esz0:, None] == t_iota[None, :], axis=0,
                   dtype=jnp.int32)
    cnt2 = jnp.stack([cnt0, cnt1])
    pcnt2 = jnp.maximum((cnt2 + ec - 1) // ec, 1) * ec
    pstart2 = jnp.concatenate(
        [jnp.zeros((2, 1), jnp.int32),
         jnp.cumsum(pcnt2, axis=1).astype(jnp.int32)], axis=1)

    lb = 14
    key_real = (core << 30) | (tile << 23) | (ldst_e << lb) | src

    pads = []
    for k in range(2):
        ncand = spc - (esz0 if k == 0 else esz1)
        ci = jnp.arange(ncand, dtype=jnp.int32)
        t = ci // ec
        i = ci - t * ec
        need = jnp.repeat(pcnt2[k] - cnt2[k], ec)
        need = jnp.concatenate(
            [need, jnp.zeros((max(ncand - nt * ec, 0),), jnp.int32)])[:ncand]
        valid = (t < nt) & (i < need)
        key = jnp.where(valid,
                        (k << 30) | (t << 23) | (ti << lb) | i,
                        (k << 30) | (nt << 23) | (ti << lb)
                        | (ci & ((1 << lb) - 1)))
        pads.append(key)
    keys_s = jnp.sort(jnp.concatenate([key_real, pads[0], pads[1]]))

    srcp = keys_s & ((1 << lb) - 1)
    ldst3 = ((keys_s >> lb) & 511).reshape(2 * nch, 1, ec)

    jslot = (jnp.arange(nch, dtype=jnp.int32) * ec)[None, :, None]
    ctile2 = jnp.sum(pstart2[:, None, :] <= jslot, axis=2,
                     dtype=jnp.int32) - 1
    ctile2 = jnp.clip(ctile2, 0, nt - 1)
    chg = (ctile2[:, 1:] != ctile2[:, :-1]).astype(jnp.int32)
    one = jnp.ones((2, 1), jnp.int32)
    cfirst = jnp.concatenate([one, chg], axis=1).reshape(-1)
    clast = jnp.concatenate([chg, one], axis=1).reshape(-1)
    ctile = ctile2.reshape(-1)
    limit = jnp.take_along_axis(pstart2[:, :nt] + cnt2, jnp.clip(ctile2, 0, nt - 1),
                                axis=1)
    cactive = ((jnp.arange(nch, dtype=jnp.int32) * ec)[None, :] < limit
               ).astype(jnp.int32).reshape(-1)

    cd = jnp.bfloat16
    w1 = jnp.concatenate([w1_l.T, w1_r.T], axis=1).astype(cd)
    b1r = b1.reshape(1, dh).astype(jnp.float32)
    f2 = dh
    w2 = jnp.concatenate([jnp.pad(w2_l.T, ((0, 0), (0, f2 - dout))),
                          w2_r.T], axis=1).astype(cd)
    b2r = b2.reshape(1, dout).astype(jnp.float32)

    tp = 512
    xl1, xr1 = pl.pallas_call(
        functools.partial(_proj_kernel, f=dh),
        out_shape=(jax.ShapeDtypeStruct((n, dh), jnp.float32),
                   jax.ShapeDtypeStruct((n, dh), jnp.float32)),
        grid=(n // tp,),
        in_specs=[pl.BlockSpec((tp, din), lambda i: (i, 0)),
                  pl.BlockSpec((din, 2 * dh), lambda i: (0, 0)),
                  pl.BlockSpec((1, dh), lambda i: (0, 0))],
        out_specs=(pl.BlockSpec((tp, dh), lambda i: (i, 0)),
                   pl.BlockSpec((tp, dh), lambda i: (i, 0))),
        compiler_params=pltpu.CompilerParams(
            dimension_semantics=("parallel",)),
    )(x.astype(cd), w1, b1r)

    vlim = 48 * 1024 * 1024
    part1, pcn1 = _sparse_agg(xl1, ctile, cfirst, clast, cactive, srcp, ldst3,
                              n=n, ec=ec, ti=ti, nch=nch, dw=dh, vlim=vlim,
                              with_cnt=True)
    tc = 512
    xl2, xr2 = pl.pallas_call(
        functools.partial(_comb_mid_kernel, f=f2),
        out_shape=(jax.ShapeDtypeStruct((n, f2), jnp.float32),
                   jax.ShapeDtypeStruct((n, dout), jnp.float32)),
        grid=(n // tc,),
        in_specs=[pl.BlockSpec((2, tc, dh), lambda i: (0, i, 0)),
                  pl.BlockSpec((2, tc, 1), lambda i: (0, i, 0)),
                  pl.BlockSpec((tc, dh), lambda i: (i, 0)),
                  pl.BlockSpec((dh, f2 + dout), lambda i: (0, 0)),
                  pl.BlockSpec((1, dout), lambda i: (0, 0))],
        out_specs=(pl.BlockSpec((tc, f2), lambda i: (i, 0)),
                   pl.BlockSpec((tc, dout), lambda i: (i, 0))),
        compiler_params=pltpu.CompilerParams(
            dimension_semantics=("parallel",)),
    )(part1, pcn1, xr1, w2, b2r)

    part2, = _sparse_agg(xl2, ctile, cfirst, clast, cactive, srcp, ldst3,
                         n=n, ec=ec, ti=ti, nch=nch, dw=dout, vlim=vlim,
                         with_cnt=False)
    out = pl.pallas_call(
        _comb_out_kernel,
        out_shape=jax.ShapeDtypeStruct((n, dout), jnp.float32),
        grid=(n // tc,),
        in_specs=[pl.BlockSpec((2, tc, dout), lambda i: (0, i, 0)),
                  pl.BlockSpec((2, tc, 1), lambda i: (0, i, 0)),
                  pl.BlockSpec((tc, dout), lambda i: (i, 0))],
        out_specs=pl.BlockSpec((tc, dout), lambda i: (i, 0)),
        compiler_params=pltpu.CompilerParams(
            dimension_semantics=("parallel",)),
    )(part2, pcn1, xr2)

    return out


# speedup vs baseline: 1.1294x; 1.0061x over previous
import functools

import jax
import jax.numpy as jnp
from jax.experimental import pallas as pl
from jax.experimental.pallas import tpu as pltpu


def _proj_kernel(x_ref, w_ref, b_ref, xl_ref, xr_ref, *, f):
    y = jnp.dot(x_ref[...], w_ref[...], preferred_element_type=jnp.float32)
    xl_ref[...] = y[:, :f]
    xr_ref[...] = (y[:, f:] + b_ref[...]).astype(xr_ref.dtype)


def _agg_part_kernel(ctile_ref, cfirst_ref, clast_ref, cactive_ref, srcp_ref,
                     ldst_ref, xl_ref, *refs, ec, ti, nch, dw, with_cnt):
    if with_cnt:
        part_ref, pcnt_ref, msgs_ref, acc_ref, dcnt_ref = refs
    else:
        part_ref, msgs_ref, acc_ref = refs
        pcnt_ref = dcnt_ref = None
    k = pl.program_id(0)
    j = pl.program_id(1)
    c = k * nch + j

    @pl.when(cfirst_ref[c] == 1)
    def _():
        acc_ref[...] = jnp.zeros_like(acc_ref)
        if with_cnt:
            dcnt_ref[...] = jnp.zeros_like(dcnt_ref)

    @pl.when(cactive_ref[c] == 1)
    def _():
        base = c * ec
        for mi in range(ec):
            idx = srcp_ref[base + mi]
            msgs_ref[pl.ds(mi, 1), :] = xl_ref[pl.ds(idx, 1), :]

        rows = jax.lax.broadcasted_iota(jnp.int32, (ti, ec), 0)
        oh = (rows == ldst_ref[...].reshape(1, ec)).astype(jnp.bfloat16)
        acc_ref[...] += jnp.dot(oh, msgs_ref[...].astype(jnp.bfloat16),
                                preferred_element_type=jnp.float32)
        if with_cnt:
            dcnt_ref[...] += jnp.sum(oh, axis=1,
                                     keepdims=True).astype(jnp.float32)

    @pl.when(clast_ref[c] == 1)
    def _():
        part_ref[0, :, :] = acc_ref[:, :dw].astype(part_ref.dtype)
        if with_cnt:
            pcnt_ref[0, :, :] = dcnt_ref[...]


def _comb_mid_kernel(p_ref, c_ref, xr_ref, w2_ref, b2_ref,
                     xl2_ref, xr2_ref, *, f):
    dinv = 1.0 / jnp.maximum(c_ref[0] + c_ref[1], 1.0)
    psum = p_ref[0].astype(jnp.float32) + p_ref[1].astype(jnp.float32)
    h = jnp.maximum(psum * dinv + xr_ref[...].astype(jnp.float32), 0.0)
    y2 = jnp.dot(h.astype(jnp.bfloat16), w2_ref[...],
                 preferred_element_type=jnp.float32)
    xl2_ref[...] = y2[:, :f]
    xr2_ref[...] = (y2[:, f:] + b2_ref[...]).astype(xr2_ref.dtype)


def _comb_out_kernel(p_ref, c_ref, xr_ref, out_ref):
    dinv = 1.0 / jnp.maximum(c_ref[0] + c_ref[1], 1.0)
    z = (p_ref[0].astype(jnp.float32)
         + p_ref[1].astype(jnp.float32)) * dinv + xr_ref[...].astype(jnp.float32)
    m = jnp.max(z, axis=-1, keepdims=True)
    lse = jnp.log(jnp.sum(jnp.exp(z - m), axis=-1, keepdims=True)) + m
    out_ref[...] = z - lse


def _sparse_agg(xl, ctile, cfirst, clast, cactive, srcp, ldst3, *, n, ec, ti, nch, dw,
                vlim, with_cnt):
    f = xl.shape[1]
    out_shape = [jax.ShapeDtypeStruct((2, n, dw), jnp.bfloat16)]
    out_specs = [pl.BlockSpec((1, ti, dw),
                              lambda k, j, ct, cf, cl, ca, sp: (k, ct[k * nch + j], 0))]
    scratch = [pltpu.VMEM((ec, f), jnp.float32),
               pltpu.VMEM((ti, f), jnp.float32)]
    if with_cnt:
        out_shape.append(jax.ShapeDtypeStruct((2, n, 1), jnp.float32))
        out_specs.append(pl.BlockSpec(
            (1, ti, 1), lambda k, j, ct, cf, cl, ca, sp: (k, ct[k * nch + j], 0)))
        scratch.append(pltpu.VMEM((ti, 1), jnp.float32))
    return pl.pallas_call(
        functools.partial(_agg_part_kernel, ec=ec, ti=ti, nch=nch, dw=dw,
                          with_cnt=with_cnt),
        out_shape=tuple(out_shape),
        grid_spec=pltpu.PrefetchScalarGridSpec(
            num_scalar_prefetch=5,
            grid=(2, nch),
            in_specs=[
                pl.BlockSpec((1, 1, ec),
                             lambda k, j, ct, cf, cl, ca, sp: (k * nch + j, 0, 0)),
                pl.BlockSpec((n, f), lambda k, j, ct, cf, cl, ca, sp: (0, 0)),
            ],
            out_specs=tuple(out_specs),
            scratch_shapes=scratch,
        ),
        compiler_params=pltpu.CompilerParams(
            dimension_semantics=("parallel", "arbitrary"),
            vmem_limit_bytes=vlim),
    )(ctile, cfirst, clast, cactive, srcp, ldst3, xl)


def kernel(x, edge_index, w1_l, w1_r, b1, w2_l, w2_r, b2):
    n, din = x.shape
    dh = w1_l.shape[0]
    dout = w2_l.shape[0]
    e = edge_index.shape[1]

    ti = 256
    ec = 256
    nt = n // ti
    esz0 = e // 2
    esz1 = e - esz0
    nch = (esz1 + ec - 1) // ec + nt
    spc = nch * ec

    src, dst = edge_index[0], edge_index[1]
    src = src.astype(jnp.int32)
    dst = dst.astype(jnp.int32)

    tile = dst // ti
    ldst_e = dst - tile * ti
    core = (jnp.arange(e, dtype=jnp.int32) >= esz0).astype(jnp.int32)

    t_iota = jnp.arange(nt, dtype=jnp.int32)
    cnt0 = jnp.sum(tile[:esz0, None] == t_iota[None, :], axis=0,
                   dtype=jnp.int32)
    cnt1 = jnp.sum(tile[esz0:, None] == t_iota[None, :], axis=0,
                   dtype=jnp.int32)
    cnt2 = jnp.stack([cnt0, cnt1])
    pcnt2 = jnp.maximum((cnt2 + ec - 1) // ec, 1) * ec
    pstart2 = jnp.concatenate(
        [jnp.zeros((2, 1), jnp.int32),
         jnp.cumsum(pcnt2, axis=1).astype(jnp.int32)], axis=1)

    lb = 14
    key_real = (core << 30) | (tile << 23) | (ldst_e << lb) | src

    pads = []
    for k in range(2):
        ncand = spc - (esz0 if k == 0 else esz1)
        ci = jnp.arange(ncand, dtype=jnp.int32)
        t = ci // ec
        i = ci - t * ec
        need = jnp.repeat(pcnt2[k] - cnt2[k], ec)
        need = jnp.concatenate(
            [need, jnp.zeros((max(ncand - nt * ec, 0),), jnp.int32)])[:ncand]
        valid = (t < nt) & (i < need)
        key = jnp.where(valid,
                        (k << 30) | (t << 23) | (ti << lb) | i,
                        (k << 30) | (nt << 23) | (ti << lb)
                        | (ci & ((1 << lb) - 1)))
        pads.append(key)
    keys_s = jnp.sort(jnp.concatenate([key_real, pads[0], pads[1]]))

    srcp = keys_s & ((1 << lb) - 1)
    ldst3 = ((keys_s >> lb) & 511).reshape(2 * nch, 1, ec)

    jslot = (jnp.arange(nch, dtype=jnp.int32) * ec)[None, :, None]
    ctile2 = jnp.sum(pstart2[:, None, :] <= jslot, axis=2,
                     dtype=jnp.int32) - 1
    ctile2 = jnp.clip(ctile2, 0, nt - 1)
    chg = (ctile2[:, 1:] != ctile2[:, :-1]).astype(jnp.int32)
    one = jnp.ones((2, 1), jnp.int32)
    cfirst = jnp.concatenate([one, chg], axis=1).reshape(-1)
    clast = jnp.concatenate([chg, one], axis=1).reshape(-1)
    ctile = ctile2.reshape(-1)
    limit = jnp.take_along_axis(pstart2[:, :nt] + cnt2, jnp.clip(ctile2, 0, nt - 1),
                                axis=1)
    cactive = ((jnp.arange(nch, dtype=jnp.int32) * ec)[None, :] < limit
               ).astype(jnp.int32).reshape(-1)

    cd = jnp.bfloat16
    w1 = jnp.concatenate([w1_l.T, w1_r.T], axis=1).astype(cd)
    b1r = b1.reshape(1, dh).astype(jnp.float32)
    f2 = dh
    w2 = jnp.concatenate([jnp.pad(w2_l.T, ((0, 0), (0, f2 - dout))),
                          w2_r.T], axis=1).astype(cd)
    b2r = b2.reshape(1, dout).astype(jnp.float32)

    tp = 512
    xl1, xr1 = pl.pallas_call(
        functools.partial(_proj_kernel, f=dh),
        out_shape=(jax.ShapeDtypeStruct((n, dh), jnp.float32),
                   jax.ShapeDtypeStruct((n, dh), jnp.bfloat16)),
        grid=(n // tp,),
        in_specs=[pl.BlockSpec((tp, din), lambda i: (i, 0)),
                  pl.BlockSpec((din, 2 * dh), lambda i: (0, 0)),
                  pl.BlockSpec((1, dh), lambda i: (0, 0))],
        out_specs=(pl.BlockSpec((tp, dh), lambda i: (i, 0)),
                   pl.BlockSpec((tp, dh), lambda i: (i, 0))),
        compiler_params=pltpu.CompilerParams(
            dimension_semantics=("parallel",)),
    )(x.astype(cd), w1, b1r)

    vlim = 48 * 1024 * 1024
    part1, pcn1 = _sparse_agg(xl1, ctile, cfirst, clast, cactive, srcp, ldst3,
                              n=n, ec=ec, ti=ti, nch=nch, dw=dh, vlim=vlim,
                              with_cnt=True)
    tc = 512
    xl2, xr2 = pl.pallas_call(
        functools.partial(_comb_mid_kernel, f=f2),
        out_shape=(jax.ShapeDtypeStruct((n, f2), jnp.float32),
                   jax.ShapeDtypeStruct((n, dout), jnp.bfloat16)),
        grid=(n // tc,),
        in_specs=[pl.BlockSpec((2, tc, dh), lambda i: (0, i, 0)),
                  pl.BlockSpec((2, tc, 1), lambda i: (0, i, 0)),
                  pl.BlockSpec((tc, dh), lambda i: (i, 0)),
                  pl.BlockSpec((dh, f2 + dout), lambda i: (0, 0)),
                  pl.BlockSpec((1, dout), lambda i: (0, 0))],
        out_specs=(pl.BlockSpec((tc, f2), lambda i: (i, 0)),
                   pl.BlockSpec((tc, dout), lambda i: (i, 0))),
        compiler_params=pltpu.CompilerParams(
            dimension_semantics=("parallel",)),
    )(part1, pcn1, xr1, w2, b2r)

    part2, = _sparse_agg(xl2, ctile, cfirst, clast, cactive, srcp, ldst3,
                         n=n, ec=ec, ti=ti, nch=nch, dw=dout, vlim=vlim,
                         with_cnt=False)
    out = pl.pallas_call(
        _comb_out_kernel,
        out_shape=jax.ShapeDtypeStruct((n, dout), jnp.float32),
        grid=(n // tc,),
        in_specs=[pl.BlockSpec((2, tc, dout), lambda i: (0, i, 0)),
                  pl.BlockSpec((2, tc, 1), lambda i: (0, i, 0)),
                  pl.BlockSpec((tc, dout), lambda i: (i, 0))],
        out_specs=pl.BlockSpec((tc, dout), lambda i: (i, 0)),
        compiler_params=pltpu.CompilerParams(
            dimension_semantics=("parallel",)),
    )(part2, pcn1, xr2)

    return out


# speedup vs baseline: 1.1933x; 1.0566x over previous
import functools

import jax
import jax.numpy as jnp
from jax.experimental import pallas as pl
from jax.experimental.pallas import tpu as pltpu


def _proj_kernel(x_ref, w_ref, b_ref, xl_ref, xr_ref, *, f):
    y = jnp.dot(x_ref[...].astype(w_ref.dtype), w_ref[...],
                preferred_element_type=jnp.float32)
    xl_ref[...] = y[:, :f]
    xr_ref[...] = (y[:, f:] + b_ref[...]).astype(xr_ref.dtype)


def _gather_scatter(c, cfirst_ref, cactive_ref, srcp_ref, ldst_ref, xl_ref,
                    msgs_ref, acc_ref, dcnt_ref, *, ec, ti):
    @pl.when(cfirst_ref[c] == 1)
    def _():
        acc_ref[...] = jnp.zeros_like(acc_ref)
        if dcnt_ref is not None:
            dcnt_ref[...] = jnp.zeros_like(dcnt_ref)

    @pl.when(cactive_ref[c] == 1)
    def _():
        base = c * ec
        for mi in range(ec):
            idx = srcp_ref[base + mi]
            msgs_ref[pl.ds(mi, 1), :] = xl_ref[pl.ds(idx, 1), :]

        rows = jax.lax.broadcasted_iota(jnp.int32, (ti, ec), 0)
        oh = (rows == ldst_ref[...].reshape(1, ec)).astype(jnp.bfloat16)
        acc_ref[...] += jnp.dot(oh, msgs_ref[...].astype(jnp.bfloat16),
                                preferred_element_type=jnp.float32)
        if dcnt_ref is not None:
            dcnt_ref[...] += jnp.sum(oh, axis=1,
                                     keepdims=True).astype(jnp.float32)


def _agg_mid_kernel(ctile_ref, cfirst_ref, clast_ref, cactive_ref, srcp_ref,
                    ldst_ref, xl_ref, xr_ref, w2_ref, b2_ref,
                    xl2_ref, xr2_ref, cnt_ref, msgs_ref, acc_ref, dcnt_ref,
                    *, ec, ti, f):
    c = pl.program_id(0)
    _gather_scatter(c, cfirst_ref, cactive_ref, srcp_ref, ldst_ref, xl_ref,
                    msgs_ref, acc_ref, dcnt_ref, ec=ec, ti=ti)

    @pl.when(clast_ref[c] == 1)
    def _():
        dinv = 1.0 / jnp.maximum(dcnt_ref[...], 1.0)
        h = jnp.maximum(acc_ref[...] * dinv + xr_ref[...].astype(jnp.float32),
                        0.0)
        y2 = jnp.dot(h.astype(jnp.bfloat16), w2_ref[...],
                     preferred_element_type=jnp.float32)
        xl2_ref[...] = y2[:, :f]
        xr2_ref[...] = (y2[:, f:] + b2_ref[...]).astype(xr2_ref.dtype)
        cnt_ref[...] = dcnt_ref[...]


def _agg_out_kernel(ctile_ref, cfirst_ref, clast_ref, cactive_ref, srcp_ref,
                    ldst_ref, xl_ref, xr_ref, cnt_ref,
                    out_ref, msgs_ref, acc_ref, *, ec, ti, dout):
    c = pl.program_id(0)
    _gather_scatter(c, cfirst_ref, cactive_ref, srcp_ref, ldst_ref, xl_ref,
                    msgs_ref, acc_ref, None, ec=ec, ti=ti)

    @pl.when(clast_ref[c] == 1)
    def _():
        dinv = 1.0 / jnp.maximum(cnt_ref[...], 1.0)
        z = acc_ref[:, :dout] * dinv + xr_ref[...].astype(jnp.float32)
        m = jnp.max(z, axis=-1, keepdims=True)
        lse = jnp.log(jnp.sum(jnp.exp(z - m), axis=-1, keepdims=True)) + m
        out_ref[...] = z - lse


def kernel(x, edge_index, w1_l, w1_r, b1, w2_l, w2_r, b2):
    n, din = x.shape
    dh = w1_l.shape[0]
    dout = w2_l.shape[0]
    e = edge_index.shape[1]

    ti = 256
    ec = 256
    nt = n // ti
    nc = (e + ec - 1) // ec + nt
    cap = nc * ec

    src, dst = edge_index[0], edge_index[1]
    src = src.astype(jnp.int32)
    dst = dst.astype(jnp.int32)

    tile = dst // ti
    ldst_e = dst - tile * ti

    t_iota = jnp.arange(nt, dtype=jnp.int32)
    cnt_t = jnp.sum(tile[:, None] == t_iota[None, :], axis=0,
                    dtype=jnp.int32)
    pcnt = jnp.maximum((cnt_t + ec - 1) // ec, 1) * ec
    pstart = jnp.concatenate([jnp.zeros((1,), jnp.int32),
                              jnp.cumsum(pcnt).astype(jnp.int32)])

    lb = 14
    key_real = (tile << 23) | (ldst_e << lb) | src

    i_g = jnp.arange(ec, dtype=jnp.int32)[None, :]
    t_g = t_iota[:, None]
    need = (pcnt - cnt_t)[:, None]
    pad_keys = jnp.where(
        i_g < need,
        (t_g << 23) | (ti << lb) | i_g,
        (nt << 23) | (ti << lb) | (t_g * ec + i_g)).reshape(-1)
    extra = cap - e - nt * ec
    parts = [key_real, pad_keys]
    if extra > 0:
        parts.append(jnp.full((extra,), (nt << 23) | (ti << lb), jnp.int32))
    keys_s = jnp.sort(jnp.concatenate(parts))

    srcp = keys_s & ((1 << lb) - 1)
    ldst3 = ((keys_s >> lb) & 511).reshape(nc, 1, ec)

    jslot = (jnp.arange(nc, dtype=jnp.int32) * ec)[:, None]
    ctile = jnp.sum(pstart[None, :] <= jslot, axis=1, dtype=jnp.int32) - 1
    ctile = jnp.clip(ctile, 0, nt - 1)
    chg = (ctile[1:] != ctile[:-1]).astype(jnp.int32)
    one = jnp.ones((1,), jnp.int32)
    cfirst = jnp.concatenate([one, chg])
    clast = jnp.concatenate([chg, one])
    limit = jnp.take_along_axis(pstart[:nt] + cnt_t, ctile, axis=0)
    cactive = ((jnp.arange(nc, dtype=jnp.int32) * ec) < limit).astype(jnp.int32)

    cd = jnp.bfloat16
    w1 = jnp.concatenate([w1_l.T, w1_r.T], axis=1).astype(cd)
    b1r = b1.reshape(1, dh).astype(jnp.float32)
    f2 = dh
    w2 = jnp.concatenate([jnp.pad(w2_l.T, ((0, 0), (0, f2 - dout))),
                          w2_r.T], axis=1).astype(cd)
    b2r = b2.reshape(1, dout).astype(jnp.float32)

    tp = 512
    xl1, xr1 = pl.pallas_call(
        functools.partial(_proj_kernel, f=dh),
        out_shape=(jax.ShapeDtypeStruct((n, dh), jnp.float32),
                   jax.ShapeDtypeStruct((n, dh), jnp.bfloat16)),
        grid=(n // tp,),
        in_specs=[pl.BlockSpec((tp, din), lambda i: (i, 0)),
                  pl.BlockSpec((din, 2 * dh), lambda i: (0, 0)),
                  pl.BlockSpec((1, dh), lambda i: (0, 0))],
        out_specs=(pl.BlockSpec((tp, dh), lambda i: (i, 0)),
                   pl.BlockSpec((tp, dh), lambda i: (i, 0))),
        compiler_params=pltpu.CompilerParams(
            dimension_semantics=("parallel",)),
    )(x, w1, b1r)

    vlim = 48 * 1024 * 1024
    xl2, xr2, cnt1 = pl.pallas_call(
        functools.partial(_agg_mid_kernel, ec=ec, ti=ti, f=f2),
        out_shape=(jax.ShapeDtypeStruct((n, f2), jnp.float32),
                   jax.ShapeDtypeStruct((n, dout), jnp.bfloat16),
                   jax.ShapeDtypeStruct((n, 1), jnp.float32)),
        grid_spec=pltpu.PrefetchScalarGridSpec(
            num_scalar_prefetch=5,
            grid=(nc,),
            in_specs=[
                pl.BlockSpec((1, 1, ec),
                             lambda c, ct, cf, cl, ca, sp: (c, 0, 0)),
                pl.BlockSpec((n, dh), lambda c, ct, cf, cl, ca, sp: (0, 0)),
                pl.BlockSpec((ti, dh),
                             lambda c, ct, cf, cl, ca, sp: (ct[c], 0)),
                pl.BlockSpec((dh, f2 + dout),
                             lambda c, ct, cf, cl, ca, sp: (0, 0)),
                pl.BlockSpec((1, dout), lambda c, ct, cf, cl, ca, sp: (0, 0)),
            ],
            out_specs=(
                pl.BlockSpec((ti, f2),
                             lambda c, ct, cf, cl, ca, sp: (ct[c], 0)),
                pl.BlockSpec((ti, dout),
                             lambda c, ct, cf, cl, ca, sp: (ct[c], 0)),
                pl.BlockSpec((ti, 1),
                             lambda c, ct, cf, cl, ca, sp: (ct[c], 0)),
            ),
            scratch_shapes=[pltpu.VMEM((ec, dh), jnp.float32),
                            pltpu.VMEM((ti, dh), jnp.float32),
                            pltpu.VMEM((ti, 1), jnp.float32)],
        ),
        compiler_params=pltpu.CompilerParams(
            dimension_semantics=("arbitrary",),
            vmem_limit_bytes=vlim),
    )(ctile, cfirst, clast, cactive, srcp, ldst3, xl1, xr1, w2, b2r)

    out = pl.pallas_call(
        functools.partial(_agg_out_kernel, ec=ec, ti=ti, dout=dout),
        out_shape=jax.ShapeDtypeStruct((n, dout), jnp.float32),
        grid_spec=pltpu.PrefetchScalarGridSpec(
            num_scalar_prefetch=5,
            grid=(nc,),
            in_specs=[
                pl.BlockSpec((1, 1, ec),
                             lambda c, ct, cf, cl, ca, sp: (c, 0, 0)),
                pl.BlockSpec((n, f2), lambda c, ct, cf, cl, ca, sp: (0, 0)),
                pl.BlockSpec((ti, dout),
                             lambda c, ct, cf, cl, ca, sp: (ct[c], 0)),
                pl.BlockSpec((ti, 1),
                             lambda c, ct, cf, cl, ca, sp: (ct[c], 0)),
            ],
            out_specs=pl.BlockSpec((ti, dout),
                                   lambda c, ct, cf, cl, ca, sp: (ct[c], 0)),
            scratch_shapes=[pltpu.VMEM((ec, f2), jnp.float32),
                            pltpu.VMEM((ti, f2), jnp.float32)],
        ),
        compiler_params=pltpu.CompilerParams(
            dimension_semantics=("arbitrary",),
            vmem_limit_bytes=vlim),
    )(ctile, cfirst, clast, cactive, srcp, ldst3, xl2, xr2, cnt1)

    return out


# speedup vs baseline: 1.3281x; 1.1130x over previous
import functools

import jax
import jax.numpy as jnp
from jax.experimental import pallas as pl
from jax.experimental.pallas import tpu as pltpu


def _proj_kernel(x_ref, w_ref, b_ref, xl_ref, xr_ref, *, f):
    y = jnp.dot(x_ref[...].astype(w_ref.dtype), w_ref[...],
                preferred_element_type=jnp.float32)
    xl_ref[...] = y[:, :f]
    xr_ref[...] = (y[:, f:] + b_ref[...]).astype(xr_ref.dtype)


def _gather_scatter(c, cfirst_ref, cactive_ref, srcp_ref, ldst_ref, xl_ref,
                    msgs_ref, acc_ref, dcnt_ref, *, ec, ti):
    @pl.when(cfirst_ref[c] == 1)
    def _():
        acc_ref[...] = jnp.zeros_like(acc_ref)
        if dcnt_ref is not None:
            dcnt_ref[...] = jnp.zeros_like(dcnt_ref)

    @pl.when(cactive_ref[c] == 1)
    def _():
        base = c * ec
        for mi in range(ec):
            idx = srcp_ref[base + mi]
            msgs_ref[pl.ds(mi, 1), :] = xl_ref[pl.ds(idx, 1), :]

        rows = jax.lax.broadcasted_iota(jnp.int32, (ti, ec), 0)
        oh = (rows == ldst_ref[...].reshape(1, ec)).astype(jnp.bfloat16)
        acc_ref[...] += jnp.dot(oh, msgs_ref[...].astype(jnp.bfloat16),
                                preferred_element_type=jnp.float32)
        if dcnt_ref is not None:
            dcnt_ref[...] += jnp.sum(oh, axis=1,
                                     keepdims=True).astype(jnp.float32)


def _agg_mid_kernel(ctile_ref, cfirst_ref, clast_ref, cactive_ref, srcp_ref,
                    ldst_ref, xl_ref, xr_ref, w2_ref, b2_ref,
                    xl2_ref, xr2_ref, cnt_ref, msgs_ref, acc_ref, dcnt_ref,
                    *, ec, ti, f):
    c = pl.program_id(0)
    _gather_scatter(c, cfirst_ref, cactive_ref, srcp_ref, ldst_ref, xl_ref,
                    msgs_ref, acc_ref, dcnt_ref, ec=ec, ti=ti)

    @pl.when(clast_ref[c] == 1)
    def _():
        dinv = 1.0 / jnp.maximum(dcnt_ref[...], 1.0)
        h = jnp.maximum(acc_ref[...] * dinv + xr_ref[...].astype(jnp.float32),
                        0.0)
        y2 = jnp.dot(h.astype(jnp.bfloat16), w2_ref[...],
                     preferred_element_type=jnp.float32)
        xl2_ref[...] = y2[:, :f]
        xr2_ref[...] = (y2[:, f:] + b2_ref[...]).astype(xr2_ref.dtype)
        cnt_ref[...] = dcnt_ref[...]


def _agg_out_kernel(ctile_ref, cfirst_ref, clast_ref, cactive_ref, srcp_ref,
                    ldst_ref, xl_ref, xr_ref, cnt_ref,
                    out_ref, msgs_ref, acc_ref, *, ec, ti, dout):
    c = pl.program_id(0)
    _gather_scatter(c, cfirst_ref, cactive_ref, srcp_ref, ldst_ref, xl_ref,
                    msgs_ref, acc_ref, None, ec=ec, ti=ti)

    @pl.when(clast_ref[c] == 1)
    def _():
        dinv = 1.0 / jnp.maximum(cnt_ref[...], 1.0)
        z = acc_ref[:, :dout] * dinv + xr_ref[...].astype(jnp.float32)
        m = jnp.max(z, axis=-1, keepdims=True)
        lse = jnp.log(jnp.sum(jnp.exp(z - m), axis=-1, keepdims=True)) + m
        out_ref[...] = z - lse


def kernel(x, edge_index, w1_l, w1_r, b1, w2_l, w2_r, b2):
    n, din = x.shape
    dh = w1_l.shape[0]
    dout = w2_l.shape[0]
    e = edge_index.shape[1]

    ti = 512
    ec = 512
    nt = n // ti
    nc = (e + ec - 1) // ec + nt
    cap = nc * ec

    src, dst = edge_index[0], edge_index[1]
    src = src.astype(jnp.int32)
    dst = dst.astype(jnp.int32)

    tile = dst // ti
    ldst_e = dst - tile * ti

    t_iota = jnp.arange(nt, dtype=jnp.int32)
    cnt_t = jnp.sum(tile[:, None] == t_iota[None, :], axis=0,
                    dtype=jnp.int32)
    pcnt = jnp.maximum((cnt_t + ec - 1) // ec, 1) * ec
    pstart = jnp.concatenate([jnp.zeros((1,), jnp.int32),
                              jnp.cumsum(pcnt).astype(jnp.int32)])

    lb = 14
    key_real = (tile << 24) | (ldst_e << lb) | src

    i_g = jnp.arange(ec, dtype=jnp.int32)[None, :]
    t_g = t_iota[:, None]
    need = (pcnt - cnt_t)[:, None]
    pad_keys = jnp.where(
        i_g < need,
        (t_g << 24) | (ti << lb) | i_g,
        (nt << 24) | (ti << lb) | ((t_g * ec + i_g) & ((1 << lb) - 1))
    ).reshape(-1)
    extra = cap - e - nt * ec
    parts = [key_real, pad_keys]
    if extra > 0:
        parts.append(jnp.full((extra,), (nt << 24) | (ti << lb), jnp.int32))
    keys_s = jnp.sort(jnp.concatenate(parts))

    srcp = keys_s & ((1 << lb) - 1)
    ldst3 = ((keys_s >> lb) & 1023).reshape(nc, 1, ec)

    jslot = (jnp.arange(nc, dtype=jnp.int32) * ec)[:, None]
    ctile = jnp.sum(pstart[None, :] <= jslot, axis=1, dtype=jnp.int32) - 1
    ctile = jnp.clip(ctile, 0, nt - 1)
    chg = (ctile[1:] != ctile[:-1]).astype(jnp.int32)
    one = jnp.ones((1,), jnp.int32)
    cfirst = jnp.concatenate([one, chg])
    clast = jnp.concatenate([chg, one])
    limit = jnp.take_along_axis(pstart[:nt] + cnt_t, ctile, axis=0)
    cactive = ((jnp.arange(nc, dtype=jnp.int32) * ec) < limit).astype(jnp.int32)

    cd = jnp.bfloat16
    w1 = jnp.concatenate([w1_l.T, w1_r.T], axis=1).astype(cd)
    b1r = b1.reshape(1, dh).astype(jnp.float32)
    f2 = dh
    w2 = jnp.concatenate([jnp.pad(w2_l.T, ((0, 0), (0, f2 - dout))),
                          w2_r.T], axis=1).astype(cd)
    b2r = b2.reshape(1, dout).astype(jnp.float32)

    tp = 512
    xl1, xr1 = pl.pallas_call(
        functools.partial(_proj_kernel, f=dh),
        out_shape=(jax.ShapeDtypeStruct((n, dh), jnp.float32),
                   jax.ShapeDtypeStruct((n, dh), jnp.bfloat16)),
        grid=(n // tp,),
        in_specs=[pl.BlockSpec((tp, din), lambda i: (i, 0)),
                  pl.BlockSpec((din, 2 * dh), lambda i: (0, 0)),
                  pl.BlockSpec((1, dh), lambda i: (0, 0))],
        out_specs=(pl.BlockSpec((tp, dh), lambda i: (i, 0)),
                   pl.BlockSpec((tp, dh), lambda i: (i, 0))),
        compiler_params=pltpu.CompilerParams(
            dimension_semantics=("parallel",)),
    )(x, w1, b1r)

    vlim = 48 * 1024 * 1024
    xl2, xr2, cnt1 = pl.pallas_call(
        functools.partial(_agg_mid_kernel, ec=ec, ti=ti, f=f2),
        out_shape=(jax.ShapeDtypeStruct((n, f2), jnp.float32),
                   jax.ShapeDtypeStruct((n, dout), jnp.bfloat16),
                   jax.ShapeDtypeStruct((n, 1), jnp.float32)),
        grid_spec=pltpu.PrefetchScalarGridSpec(
            num_scalar_prefetch=5,
            grid=(nc,),
            in_specs=[
                pl.BlockSpec((1, 1, ec),
                             lambda c, ct, cf, cl, ca, sp: (c, 0, 0)),
                pl.BlockSpec((n, dh), lambda c, ct, cf, cl, ca, sp: (0, 0)),
                pl.BlockSpec((ti, dh),
                             lambda c, ct, cf, cl, ca, sp: (ct[c], 0)),
                pl.BlockSpec((dh, f2 + dout),
                             lambda c, ct, cf, cl, ca, sp: (0, 0)),
                pl.BlockSpec((1, dout), lambda c, ct, cf, cl, ca, sp: (0, 0)),
            ],
            out_specs=(
                pl.BlockSpec((ti, f2),
                             lambda c, ct, cf, cl, ca, sp: (ct[c], 0)),
                pl.BlockSpec((ti, dout),
                             lambda c, ct, cf, cl, ca, sp: (ct[c], 0)),
                pl.BlockSpec((ti, 1),
                             lambda c, ct, cf, cl, ca, sp: (ct[c], 0)),
            ),
            scratch_shapes=[pltpu.VMEM((ec, dh), jnp.float32),
                            pltpu.VMEM((ti, dh), jnp.float32),
                            pltpu.VMEM((ti, 1), jnp.float32)],
        ),
        compiler_params=pltpu.CompilerParams(
            dimension_semantics=("arbitrary",),
            vmem_limit_bytes=vlim),
    )(ctile, cfirst, clast, cactive, srcp, ldst3, xl1, xr1, w2, b2r)

    out = pl.pallas_call(
        functools.partial(_agg_out_kernel, ec=ec, ti=ti, dout=dout),
        out_shape=jax.ShapeDtypeStruct((n, dout), jnp.float32),
        grid_spec=pltpu.PrefetchScalarGridSpec(
            num_scalar_prefetch=5,
            grid=(nc,),
            in_specs=[
                pl.BlockSpec((1, 1, ec),
                             lambda c, ct, cf, cl, ca, sp: (c, 0, 0)),
                pl.BlockSpec((n, f2), lambda c, ct, cf, cl, ca, sp: (0, 0)),
                pl.BlockSpec((ti, dout),
                             lambda c, ct, cf, cl, ca, sp: (ct[c], 0)),
                pl.BlockSpec((ti, 1),
                             lambda c, ct, cf, cl, ca, sp: (ct[c], 0)),
            ],
            out_specs=pl.BlockSpec((ti, dout),
                                   lambda c, ct, cf, cl, ca, sp: (ct[c], 0)),
            scratch_shapes=[pltpu.VMEM((ec, f2), jnp.float32),
                            pltpu.VMEM((ti, f2), jnp.float32)],
        ),
        compiler_params=pltpu.CompilerParams(
            dimension_semantics=("arbitrary",),
            vmem_limit_bytes=vlim),
    )(ctile, cfirst, clast, cactive, srcp, ldst3, xl2, xr2, cnt1)

    return out
